# baseline (speedup 1.0000x reference)
import jax
import jax.numpy as jnp
from jax import lax
from jax.experimental import pallas as pl

_N = 10000
_E = 320000
_D = 128
_T = 4


def _tc_pre(x, W1, b1):
    n, d = x.shape
    t = W1.shape[0]
    bn = 1000

    def body(x_ref, w1_ref, b1_ref, a_ref, b_ref):
        xb = x_ref[...]
        a_ref[0] = (
            jnp.dot(xb, w1_ref[0, :d, :], preferred_element_type=jnp.float32)
            + b1_ref[0]
        ).astype(jnp.bfloat16)
        b_ref[0] = jnp.dot(
            xb, w1_ref[0, d:, :], preferred_element_type=jnp.float32
        ).astype(jnp.bfloat16)

    grid = (t, n // bn)
    out = pl.pallas_call(
        body,
        grid=grid,
        in_specs=[
            pl.BlockSpec((bn, d), lambda ti, i: (i, 0)),
            pl.BlockSpec((1, 2 * d, d), lambda ti, i: (ti, 0, 0)),
            pl.BlockSpec((1, 1, d), lambda ti, i: (ti, 0, 0)),
        ],
        out_specs=[
            pl.BlockSpec((1, bn, d), lambda ti, i: (ti, i, 0)),
            pl.BlockSpec((1, bn, d), lambda ti, i: (ti, i, 0)),
        ],
        out_shape=[
            jax.ShapeDtypeStruct((t, n, d), jnp.bfloat16),
            jax.ShapeDtypeStruct((t, n, d), jnp.bfloat16),
        ],
    )(x, W1, b1.reshape(t, 1, d))
    return out


def _tc_post(x, S2, W_self, b_self, W2, b2, ln_gamma, ln_beta):
    n, d = x.shape
    t = S2.shape[0]
    da = S2.shape[2]
    bn = 1000

    def body(x_ref, s_ref, ws_ref, bs_ref, w2_ref, b2_ref, g_ref,
             be_ref, o_ref):
        acc = (
            jnp.dot(x_ref[...], ws_ref[...], preferred_element_type=jnp.float32)
            + bs_ref[0]
        )
        for ti in range(t):
            acc = acc + jnp.dot(
                s_ref[ti, :, :d], w2_ref[ti],
                preferred_element_type=jnp.float32,
            )
            acc = acc + s_ref[ti, :, d:d + 1] * b2_ref[ti]
        acc = jnp.maximum(acc, 0.0)
        mean = jnp.mean(acc, axis=-1, keepdims=True)
        cen = acc - mean
        var = jnp.mean(cen * cen, axis=-1, keepdims=True)
        o_ref[...] = cen * lax.rsqrt(var + 1e-5) * g_ref[0] + be_ref[0]

    out = pl.pallas_call(
        body,
        grid=(n // bn,),
        in_specs=[
            pl.BlockSpec((bn, d), lambda i: (i, 0)),
            pl.BlockSpec((t, bn, da), lambda i: (0, i, 0)),
            pl.BlockSpec((d, d), lambda i: (0, 0)),
            pl.BlockSpec((1, d), lambda i: (0, 0)),
            pl.BlockSpec((t, d, d), lambda i: (0, 0, 0)),
            pl.BlockSpec((t, d), lambda i: (0, 0)),
            pl.BlockSpec((1, d), lambda i: (0, 0)),
            pl.BlockSpec((1, d), lambda i: (0, 0)),
        ],
        out_specs=pl.BlockSpec((bn, d), lambda i: (i, 0)),
        out_shape=jax.ShapeDtypeStruct((n, d), jnp.float32),
    )(x, S2, W_self, b_self.reshape(1, d), W2, b2,
      ln_gamma.reshape(1, d), ln_beta.reshape(1, d))
    return out


def kernel(x, edge_index, edge_attr, W_self, b_self, W1, b1, W2, b2,
           ln_gamma, ln_beta):
    A, B = _tc_pre(x, W1, b1)
    Af = A.reshape(_T * _N, _D)
    Bf = B.reshape(_T * _N, _D)
    row = edge_index[0]
    col = edge_index[1]
    seg = edge_attr * _N + col
    h = jnp.maximum(
        jnp.take(Af, edge_attr * _N + row, axis=0).astype(jnp.float32)
        + jnp.take(Bf, seg, axis=0).astype(jnp.float32),
        0.0,
    )
    h_aug = jnp.concatenate([h, jnp.ones((_E, 8), jnp.float32)], axis=1)
    S2 = jax.ops.segment_sum(h_aug, seg, num_segments=_T * _N).reshape(
        _T, _N, _D + 8
    )
    return _tc_post(x, S2, W_self, b_self, W2, b2, ln_gamma, ln_beta)


# speedup vs baseline: 1.2290x; 1.2290x over previous
import jax
import jax.numpy as jnp
from jax import lax
from jax.experimental import pallas as pl

_N = 10000
_E = 320000
_D = 128
_T = 4


def _tc_pre(x, W1, b1):
    n, d = x.shape
    t = W1.shape[0]
    bn = 1000

    def body(x_ref, w1_ref, b1_ref, a_ref, b_ref):
        xb = x_ref[...]
        a_ref[0] = (
            jnp.dot(xb, w1_ref[0, :d, :], preferred_element_type=jnp.float32)
            + b1_ref[0]
        ).astype(jnp.bfloat16)
        b_ref[0] = jnp.dot(
            xb, w1_ref[0, d:, :], preferred_element_type=jnp.float32
        ).astype(jnp.bfloat16)

    grid = (t, n // bn)
    out = pl.pallas_call(
        body,
        grid=grid,
        in_specs=[
            pl.BlockSpec((bn, d), lambda ti, i: (i, 0)),
            pl.BlockSpec((1, 2 * d, d), lambda ti, i: (ti, 0, 0)),
            pl.BlockSpec((1, 1, d), lambda ti, i: (ti, 0, 0)),
        ],
        out_specs=[
            pl.BlockSpec((1, bn, d), lambda ti, i: (ti, i, 0)),
            pl.BlockSpec((1, bn, d), lambda ti, i: (ti, i, 0)),
        ],
        out_shape=[
            jax.ShapeDtypeStruct((t, n, d), jnp.bfloat16),
            jax.ShapeDtypeStruct((t, n, d), jnp.bfloat16),
        ],
    )(x, W1, b1.reshape(t, 1, d))
    return out


def _tc_post(x, S, CNT, W_self, b_self, W2, b2, ln_gamma, ln_beta):
    n, d = x.shape
    t = S.shape[0]
    bn = 1000

    def body(x_ref, s_ref, c_ref, ws_ref, bs_ref, w2_ref, b2_ref, g_ref,
             be_ref, o_ref):
        acc = (
            jnp.dot(x_ref[...], ws_ref[...], preferred_element_type=jnp.float32)
            + bs_ref[0]
        )
        for ti in range(t):
            acc = acc + jnp.dot(
                s_ref[ti], w2_ref[ti], preferred_element_type=jnp.float32
            )
            acc = acc + c_ref[ti, :, 0:1] * b2_ref[ti]
        acc = jnp.maximum(acc, 0.0)
        mean = jnp.mean(acc, axis=-1, keepdims=True)
        cen = acc - mean
        var = jnp.mean(cen * cen, axis=-1, keepdims=True)
        o_ref[...] = cen * lax.rsqrt(var + 1e-5) * g_ref[0] + be_ref[0]

    out = pl.pallas_call(
        body,
        grid=(n // bn,),
        in_specs=[
            pl.BlockSpec((bn, d), lambda i: (i, 0)),
            pl.BlockSpec((t, bn, d), lambda i: (0, i, 0)),
            pl.BlockSpec((t, bn, 16), lambda i: (0, i, 0)),
            pl.BlockSpec((d, d), lambda i: (0, 0)),
            pl.BlockSpec((1, d), lambda i: (0, 0)),
            pl.BlockSpec((t, d, d), lambda i: (0, 0, 0)),
            pl.BlockSpec((t, d), lambda i: (0, 0)),
            pl.BlockSpec((1, d), lambda i: (0, 0)),
            pl.BlockSpec((1, d), lambda i: (0, 0)),
        ],
        out_specs=pl.BlockSpec((bn, d), lambda i: (i, 0)),
        out_shape=jax.ShapeDtypeStruct((n, d), jnp.float32),
    )(x, S, CNT, W_self, b_self.reshape(1, d), W2, b2,
      ln_gamma.reshape(1, d), ln_beta.reshape(1, d))
    return out


def kernel(x, edge_index, edge_attr, W_self, b_self, W1, b1, W2, b2,
           ln_gamma, ln_beta):
    A, B = _tc_pre(x, W1, b1)
    Af = A.reshape(_T * _N, _D)
    Bf = B.reshape(_T * _N, _D)
    row = edge_index[0]
    col = edge_index[1]
    seg = edge_attr * _N + col
    ridx = edge_attr * _N + row
    half = _E // 2

    def _part(lo):
        segp = lax.dynamic_slice_in_dim(seg, lo, half)
        hp = jnp.maximum(
            jnp.take(Af, lax.dynamic_slice_in_dim(ridx, lo, half), axis=0)
            .astype(jnp.float32)
            + jnp.take(Bf, segp, axis=0).astype(jnp.float32),
            0.0,
        )
        sp = jax.ops.segment_sum(hp, segp, num_segments=_T * _N)
        cp = jax.ops.segment_sum(
            jnp.ones((half, 16), jnp.float32), segp, num_segments=_T * _N
        )
        return sp, cp

    s0, c0 = _part(0)
    s1, c1 = _part(half)
    S = (s0 + s1).reshape(_T, _N, _D)
    cnt = (c0 + c1).reshape(_T, _N, 16)
    return _tc_post(x, S, cnt, W_self, b_self, W2, b2, ln_gamma, ln_beta)
